# baseline (speedup 1.0000x reference)
import functools

import jax
import jax.numpy as jnp
from jax import lax
from jax.experimental import pallas as pl
from jax.experimental.pallas import tpu as pltpu
from jax.experimental.pallas import tpu_sc as plsc

CHUNK = 128
TCOLS = 512


def _transpose_table(weight_t, V, D):
    n_out_rows = V * D // CHUNK
    grid = (V + TCOLS - 1) // TCOLS
    rows_per_step = TCOLS * D // CHUNK

    def body(wt_ref, out_ref):
        x = wt_ref[...]
        y = x.T
        y3 = y.reshape(rows_per_step, CHUNK // D, D)
        out_ref[...] = jnp.concatenate(
            [y3[:, i, :] for i in range(CHUNK // D)], axis=-1
        )

    return pl.pallas_call(
        body,
        grid=(grid,),
        in_specs=[pl.BlockSpec((D, TCOLS), lambda j: (0, j))],
        out_specs=pl.BlockSpec((rows_per_step, CHUNK), lambda j: (j, 0)),
        out_shape=jax.ShapeDtypeStruct((n_out_rows, CHUNK), jnp.float32),
    )(weight_t)


def _gather_rows(idx, table, N, D, NC, NS, mesh):
    NW = NC * NS
    G, K = idx.shape[1], idx.shape[2]
    GROUP = K * CHUNK
    table2d = table.reshape(-1, D)

    @functools.partial(
        pl.kernel,
        out_type=jax.ShapeDtypeStruct((N, D), jnp.float32),
        mesh=mesh,
        scratch_types=[
            pltpu.VMEM((G, K, CHUNK), jnp.int32),
            pltpu.VMEM((2, GROUP, D), jnp.float32),
            pltpu.SemaphoreType.DMA,
            pltpu.SemaphoreType.DMA,
            pltpu.SemaphoreType.DMA,
        ],
        compiler_params=pltpu.CompilerParams(use_tc_tiling_on_sc=False),
    )
    def emb(idx_hbm, table_hbm, out_hbm, idx_v, rows_v, gsem, ssem0, ssem1):
        wid = lax.axis_index("s") * NC + lax.axis_index("c")
        base = wid * G * GROUP
        pltpu.sync_copy(idx_hbm.at[wid], idx_v)
        ssems = (ssem0, ssem1)

        def gather_group(g, b):
            descs = [
                pltpu.async_copy(
                    table_hbm.at[idx_v.at[g, k]],
                    rows_v.at[b, pl.ds(k * CHUNK, CHUNK)],
                    gsem,
                )
                for k in range(K)
            ]
            for d_ in descs:
                d_.wait()

        def fire_store(g, b):
            pltpu.async_copy(
                rows_v.at[b], out_hbm.at[pl.ds(base + g * GROUP, GROUP)], ssems[b]
            )

        def wait_store(b):
            pltpu.make_async_copy(
                rows_v.at[b], out_hbm.at[pl.ds(0, GROUP)], ssems[b]
            ).wait()

        gather_group(0, 0)
        fire_store(0, 0)
        gather_group(1, 1)
        fire_store(1, 1)

        @pl.loop(2, G, step=2)
        def _(g):
            for b in range(2):
                wait_store(b)
                gather_group(g + b, b)
                fire_store(g + b, b)

        wait_store(0)
        wait_store(1)

    return emb(idx, table2d)


def kernel(indices, weight):
    B, F = indices.shape
    V, D = weight.shape
    N = B * F

    info = plsc.get_sparse_core_info()
    NC, NS = info.num_cores, info.num_subcores
    NW = NC * NS
    per_w = N // NW
    n_chunks = per_w // CHUNK
    K = 13
    G = n_chunks // K
    assert per_w * NW == N and n_chunks * CHUNK == per_w
    assert G * K == n_chunks and G % 2 == 0

    mesh = plsc.VectorSubcoreMesh(core_axis_name="c", subcore_axis_name="s")

    flat_table = _transpose_table(weight.T, V, D).reshape(V * D)
    idx = indices.reshape(NW, G, K, CHUNK).astype(jnp.int32)
    out = _gather_rows(idx, flat_table, N, D, NC, NS, mesh)
    return out.reshape(B, F, D)


# speedup vs baseline: 1.6440x; 1.6440x over previous
import functools

import jax
import jax.numpy as jnp
from jax import lax
from jax.experimental import pallas as pl
from jax.experimental.pallas import tpu as pltpu
from jax.experimental.pallas import tpu_sc as plsc

CHUNK = 128
PAD = 128


def _gather_rows(idx, table_pad, N, D, NC, NS, mesh):
    NW = NC * NS
    G, K = idx.shape[1], idx.shape[2]
    GROUP = K * CHUNK

    @functools.partial(
        pl.kernel,
        out_type=jax.ShapeDtypeStruct((N, D), jnp.float32),
        mesh=mesh,
        scratch_types=[
            pltpu.VMEM((G, K, CHUNK), jnp.int32),
            pltpu.VMEM((2, GROUP, PAD), jnp.float32),
            pltpu.SemaphoreType.DMA,
            pltpu.SemaphoreType.DMA,
            pltpu.SemaphoreType.DMA,
        ],
        compiler_params=pltpu.CompilerParams(use_tc_tiling_on_sc=False),
    )
    def emb(idx_hbm, table_hbm, out_hbm, idx_v, rows_v, gsem, ssem0, ssem1):
        wid = lax.axis_index("s") * NC + lax.axis_index("c")
        base = wid * G * GROUP
        pltpu.sync_copy(idx_hbm.at[wid], idx_v)
        ssems = (ssem0, ssem1)

        def gather_group(g, b):
            descs = [
                pltpu.async_copy(
                    table_hbm.at[idx_v.at[g, k]],
                    rows_v.at[b, pl.ds(k * CHUNK, CHUNK)],
                    gsem,
                )
                for k in range(K)
            ]
            for d_ in descs:
                d_.wait()

        def fire_store(g, b):
            pltpu.async_copy(
                rows_v.at[b, pl.ds(0, GROUP), pl.ds(0, D)],
                out_hbm.at[pl.ds(base + g * GROUP, GROUP)],
                ssems[b],
            )

        def wait_store(b):
            pltpu.make_async_copy(
                rows_v.at[b, pl.ds(0, GROUP), pl.ds(0, D)],
                out_hbm.at[pl.ds(0, GROUP)],
                ssems[b],
            ).wait()

        gather_group(0, 0)
        fire_store(0, 0)
        gather_group(1, 1)
        fire_store(1, 1)

        @pl.loop(2, G, step=2)
        def _(g):
            for b in range(2):
                wait_store(b)
                gather_group(g + b, b)
                fire_store(g + b, b)

        wait_store(0)
        wait_store(1)

    return emb(idx, table_pad)


def kernel(indices, weight):
    B, F = indices.shape
    V, D = weight.shape
    N = B * F

    info = plsc.get_sparse_core_info()
    NC, NS = info.num_cores, info.num_subcores
    NW = NC * NS
    per_w = N // NW
    n_chunks = per_w // CHUNK
    K = 2
    G = n_chunks // K
    assert per_w * NW == N and n_chunks * CHUNK == per_w
    assert G * K == n_chunks and G % 2 == 0

    mesh = plsc.VectorSubcoreMesh(core_axis_name="c", subcore_axis_name="s")

    table_pad = jnp.pad(weight, ((0, 0), (0, PAD - D)))
    idx = indices.reshape(NW, G, K, CHUNK).astype(jnp.int32)
    out = _gather_rows(idx, table_pad, N, D, NC, NS, mesh)
    return out.reshape(B, F, D)


# speedup vs baseline: 1.9016x; 1.1567x over previous
import functools

import jax
import jax.numpy as jnp
from jax import lax
from jax.experimental import pallas as pl
from jax.experimental.pallas import tpu as pltpu
from jax.experimental.pallas import tpu_sc as plsc

CHUNK = 128


def _gather_rows(idx, table2d, N, D, NC, NS, mesh):
    NW = NC * NS
    G, K = idx.shape[1], idx.shape[2]
    GROUP = K * CHUNK

    @functools.partial(
        pl.kernel,
        out_type=jax.ShapeDtypeStruct((N, D), jnp.float32),
        mesh=mesh,
        scratch_types=[
            pltpu.VMEM((G, K, CHUNK), jnp.int32),
            pltpu.VMEM((2, GROUP, D), jnp.float32),
            pltpu.SemaphoreType.DMA,
            pltpu.SemaphoreType.DMA,
            pltpu.SemaphoreType.DMA,
        ],
        compiler_params=pltpu.CompilerParams(use_tc_tiling_on_sc=False),
    )
    def emb(idx_hbm, table_hbm, out_hbm, idx_v, rows_v, gsem, ssem0, ssem1):
        wid = lax.axis_index("s") * NC + lax.axis_index("c")
        base = wid * G * GROUP
        pltpu.sync_copy(idx_hbm.at[wid], idx_v)
        ssems = (ssem0, ssem1)

        def gather_group(g, b):
            descs = [
                pltpu.async_copy(
                    table_hbm.at[idx_v.at[g, k]],
                    rows_v.at[b, pl.ds(k * CHUNK, CHUNK)],
                    gsem,
                )
                for k in range(K)
            ]
            for d_ in descs:
                d_.wait()

        def fire_store(g, b):
            pltpu.async_copy(
                rows_v.at[b], out_hbm.at[pl.ds(base + g * GROUP, GROUP)], ssems[b]
            )

        def wait_store(b):
            pltpu.make_async_copy(
                rows_v.at[b], out_hbm.at[pl.ds(0, GROUP)], ssems[b]
            ).wait()

        gather_group(0, 0)
        fire_store(0, 0)
        gather_group(1, 1)
        fire_store(1, 1)

        @pl.loop(2, G, step=2)
        def _(g):
            for b in range(2):
                wait_store(b)
                gather_group(g + b, b)
                fire_store(g + b, b)

        wait_store(0)
        wait_store(1)

    return emb(idx, table2d)


def kernel(indices, weight):
    B, F = indices.shape
    V, D = weight.shape
    N = B * F

    info = plsc.get_sparse_core_info()
    NC, NS = info.num_cores, info.num_subcores
    NW = NC * NS
    per_w = N // NW
    n_chunks = per_w // CHUNK
    K = 13
    G = n_chunks // K
    assert per_w * NW == N and n_chunks * CHUNK == per_w
    assert G * K == n_chunks and G % 2 == 0

    mesh = plsc.VectorSubcoreMesh(core_axis_name="c", subcore_axis_name="s")

    w_lin = lax.optimization_barrier(weight.reshape(V * D // CHUNK, CHUNK))
    table2d = w_lin.reshape(V, D)
    idx = indices.reshape(NW, G, K, CHUNK).astype(jnp.int32)
    out = _gather_rows(idx, table2d, N, D, NC, NS, mesh)
    return out.reshape(B, F, D)
